# baseline (speedup 1.0000x reference)
import functools

import jax
import jax.numpy as jnp
from jax import lax
from jax.experimental import pallas as pl
from jax.experimental.pallas import tpu as pltpu
from jax.experimental.pallas import tpu_sc as plsc

B = 16384
D = 64
H1 = 32
HP = H1 // 2

_info = plsc.get_sparse_core_info()
_NC, _NS = _info.num_cores, _info.num_subcores
_NW = _NC * _NS
_BPW = B // _NW


def _proj_body(t_ref, w_ref, o_ref):
    pt = lax.dot_general(
        w_ref[...], t_ref[...], (((0,), (0,)), ((), ())),
        preferred_element_type=jnp.float32)
    bits = lax.bitcast_convert_type(pt, jnp.uint32)
    lo = (bits[:HP] + 0x8000) >> 16
    hi = (bits[HP:] + 0x8000) & jnp.uint32(0xFFFF0000)
    packed = lax.bitcast_convert_type(lo | hi, jnp.int32)
    o_ref[...] = packed.T


def _project(table_t, w):
    rows = table_t.shape[1]
    blk = 32768
    return pl.pallas_call(
        _proj_body,
        grid=(pl.cdiv(rows, blk),),
        in_specs=[
            pl.BlockSpec((D, blk), lambda i: (0, i)),
            pl.BlockSpec((D, H1), lambda i: (0, 0)),
        ],
        out_specs=pl.BlockSpec((blk, HP), lambda i: (i, 0)),
        out_shape=jax.ShapeDtypeStruct((rows, HP), jnp.int32),
        compiler_params=pltpu.CompilerParams(
            dimension_semantics=("arbitrary",),
            vmem_limit_bytes=64 * 1024 * 1024,
        ),
    )(table_t, w)


def _gather_sc(ids, ptab):
    mesh = plsc.VectorSubcoreMesh(core_axis_name="c", subcore_axis_name="s")

    @functools.partial(
        pl.kernel,
        mesh=mesh,
        out_type=jax.ShapeDtypeStruct((B, HP), jnp.int32),
        scratch_types=[
            pltpu.VMEM((_BPW,), jnp.int32),
            pltpu.VMEM((_BPW, HP), jnp.int32),
            pltpu.SemaphoreType.DMA,
        ],
    )
    def gather(idx_hbm, tab_hbm, out_hbm, idx_v, rows_v, sem):
        wid = lax.axis_index("s") * _NC + lax.axis_index("c")
        base = wid * _BPW
        pltpu.sync_copy(idx_hbm.at[pl.ds(base, _BPW)], idx_v)

        def body(g, carry):
            row = g * 16
            vec = idx_v[pl.ds(row, 16)]
            for k in range(16):
                pltpu.async_copy(tab_hbm.at[vec[k]], rows_v.at[row + k], sem)
            return carry

        lax.fori_loop(0, _BPW // 16, body, 0)
        pltpu.make_async_copy(tab_hbm.at[pl.ds(0, _BPW)], rows_v, sem).wait()
        pltpu.sync_copy(rows_v, out_hbm.at[pl.ds(base, _BPW)])

    return gather(ids, ptab)


def _unpack_lo(x):
    return lax.bitcast_convert_type(x << 16, jnp.float32)


def _unpack_hi(x):
    return lax.bitcast_convert_type(x & jnp.int32(-65536), jnp.float32)


def _tail_body(u_ref, j_ref, b1lo_ref, b1hi_ref, w2lo_ref, w2hi_ref, b2_ref,
               w3_ref, b3_ref, w4_ref, b4_ref, o_ref):
    u = u_ref[...]
    j = j_ref[...]
    xlo = jnp.maximum(_unpack_lo(u) + _unpack_lo(j) + b1lo_ref[...], 0.0)
    xhi = jnp.maximum(_unpack_hi(u) + _unpack_hi(j) + b1hi_ref[...], 0.0)
    x = xlo @ w2lo_ref[...] + xhi @ w2hi_ref[...] + b2_ref[...]
    x = jnp.maximum(x, 0.0)
    x = jnp.maximum(x @ w3_ref[...] + b3_ref[...], 0.0)
    x = jnp.maximum(x @ w4_ref[...] + b4_ref[...], 0.0)
    o_ref[...] = x


def _mlp_tail(gu, gj, b1, W2, b2, W3, b3, W4, b4):
    blk = 2048
    full = lambda s: pl.BlockSpec(s, lambda i: (0, 0))
    return pl.pallas_call(
        _tail_body,
        grid=(B // blk,),
        in_specs=[
            pl.BlockSpec((blk, HP), lambda i: (i, 0)),
            pl.BlockSpec((blk, HP), lambda i: (i, 0)),
            full((1, HP)), full((1, HP)),
            full((HP, 16)), full((HP, 16)), full((1, 16)),
            full((16, 12)), full((1, 12)),
            full((12, 1)), full((1, 1)),
        ],
        out_specs=pl.BlockSpec((blk, 1), lambda i: (i, 0)),
        out_shape=jax.ShapeDtypeStruct((B, 1), jnp.float32),
        compiler_params=pltpu.CompilerParams(
            dimension_semantics=("arbitrary",),
        ),
    )(gu, gj, b1[:HP].reshape(1, HP), b1[HP:].reshape(1, HP),
      W2[:HP], W2[HP:], b2.reshape(1, 16),
      W3, b3.reshape(1, 12), W4, b4.reshape(1, 1))


def kernel(user_ids, joke_ids, user_table, joke_table,
           W1, b1, W2, b2, W3, b3, W4, b4):
    uids = user_ids.reshape(B).astype(jnp.int32)
    jids = joke_ids.reshape(B).astype(jnp.int32)
    pj = _project(joke_table.T, W1[D:])
    gj = _gather_sc(jids, pj)
    pu = _project(user_table.T, W1[:D])
    gu = _gather_sc(uids, pu)
    return _mlp_tail(gu, gj, b1, W2, b2, W3, b3, W4, b4)


# speedup vs baseline: 1.0332x; 1.0332x over previous
import functools

import jax
import jax.numpy as jnp
from jax import lax
from jax.experimental import pallas as pl
from jax.experimental.pallas import tpu as pltpu
from jax.experimental.pallas import tpu_sc as plsc

B = 16384
D = 64
H1 = 32

_info = plsc.get_sparse_core_info()
_NC, _NS = _info.num_cores, _info.num_subcores
_NW = _NC * _NS
_BPW = B // _NW


def _proj_body(t_ref, w_ref, o_ref):
    pt = lax.dot_general(
        w_ref[...], t_ref[...], (((0,), (0,)), ((), ())),
        preferred_element_type=jnp.float32)
    o_ref[...] = pt.T


def _project(table_t, w):
    rows = table_t.shape[1]
    blk = 38912
    return pl.pallas_call(
        _proj_body,
        grid=(pl.cdiv(rows, blk),),
        in_specs=[
            pl.BlockSpec((D, blk), lambda i: (0, i)),
            pl.BlockSpec((D, H1), lambda i: (0, 0)),
        ],
        out_specs=pl.BlockSpec((blk, H1), lambda i: (i, 0)),
        out_shape=jax.ShapeDtypeStruct((rows, H1), jnp.float32),
        compiler_params=pltpu.CompilerParams(
            dimension_semantics=("arbitrary",),
            vmem_limit_bytes=64 * 1024 * 1024,
        ),
    )(table_t, w)


def _gather_sc(ids, ptab):
    mesh = plsc.VectorSubcoreMesh(core_axis_name="c", subcore_axis_name="s")

    @functools.partial(
        pl.kernel,
        mesh=mesh,
        out_type=jax.ShapeDtypeStruct((B, H1), jnp.float32),
        scratch_types=[
            pltpu.VMEM((_BPW,), jnp.int32),
            pltpu.VMEM((_BPW, H1), jnp.float32),
            pltpu.SemaphoreType.DMA,
        ],
    )
    def gather(idx_hbm, tab_hbm, out_hbm, idx_v, rows_v, sem):
        wid = lax.axis_index("s") * _NC + lax.axis_index("c")
        base = wid * _BPW
        pltpu.sync_copy(idx_hbm.at[pl.ds(base, _BPW)], idx_v)

        def body(g, carry):
            row = g * 16
            vec = idx_v[pl.ds(row, 16)]
            for k in range(16):
                pltpu.async_copy(tab_hbm.at[vec[k]], rows_v.at[row + k], sem)
            return carry

        lax.fori_loop(0, _BPW // 16, body, 0)
        pltpu.make_async_copy(tab_hbm.at[pl.ds(0, _BPW)], rows_v, sem).wait()
        pltpu.sync_copy(rows_v, out_hbm.at[pl.ds(base, _BPW)])

    return gather(ids, ptab)


def _tail_body(u_ref, j_ref, b1_ref, w2_ref, b2_ref,
               w3_ref, b3_ref, w4_ref, b4_ref, o_ref):
    x = jnp.maximum(u_ref[...] + j_ref[...] + b1_ref[...], 0.0)
    x = jnp.maximum(x @ w2_ref[...] + b2_ref[...], 0.0)
    x = jnp.maximum(x @ w3_ref[...] + b3_ref[...], 0.0)
    x = jnp.maximum(x @ w4_ref[...] + b4_ref[...], 0.0)
    o_ref[...] = x.T


def _mlp_tail(gu, gj, b1, W2, b2, W3, b3, W4, b4):
    blk = 8192
    full = lambda s: pl.BlockSpec(s, lambda i: (0, 0))
    return pl.pallas_call(
        _tail_body,
        grid=(B // blk,),
        in_specs=[
            pl.BlockSpec((blk, H1), lambda i: (i, 0)),
            pl.BlockSpec((blk, H1), lambda i: (i, 0)),
            full((1, H1)),
            full((H1, 16)), full((1, 16)),
            full((16, 12)), full((1, 12)),
            full((12, 1)), full((1, 1)),
        ],
        out_specs=pl.BlockSpec((1, blk), lambda i: (0, i)),
        out_shape=jax.ShapeDtypeStruct((1, B), jnp.float32),
        compiler_params=pltpu.CompilerParams(
            dimension_semantics=("arbitrary",),
        ),
    )(gu, gj, b1.reshape(1, H1), W2, b2.reshape(1, 16),
      W3, b3.reshape(1, 12), W4, b4.reshape(1, 1))


def kernel(user_ids, joke_ids, user_table, joke_table,
           W1, b1, W2, b2, W3, b3, W4, b4):
    uids = user_ids.reshape(B).astype(jnp.int32)
    jids = joke_ids.reshape(B).astype(jnp.int32)
    pj = _project(joke_table.T, W1[D:])
    gj = _gather_sc(jids, pj)
    pu = _project(user_table.T, W1[:D])
    gu = _gather_sc(uids, pu)
    out = _mlp_tail(gu, gj, b1, W2, b2, W3, b3, W4, b4)
    return out.reshape(B, 1)


# speedup vs baseline: 1.0343x; 1.0011x over previous
import functools

import jax
import jax.numpy as jnp
from jax import lax
from jax.experimental import pallas as pl
from jax.experimental.pallas import tpu as pltpu
from jax.experimental.pallas import tpu_sc as plsc

B = 16384
D = 64
H1 = 32

_info = plsc.get_sparse_core_info()
_NC, _NS = _info.num_cores, _info.num_subcores
_NW = _NC * _NS
_BPW = B // _NW


def _proj_body(t_ref, w_ref, o_ref):
    pt = lax.dot_general(
        w_ref[...], t_ref[...], (((0,), (0,)), ((), ())),
        preferred_element_type=jnp.float32)
    o_ref[...] = pt.T


def _project(table_t, w):
    rows = table_t.shape[1]
    blk = 38912
    return pl.pallas_call(
        _proj_body,
        grid=(pl.cdiv(rows, blk),),
        in_specs=[
            pl.BlockSpec((D, blk), lambda i: (0, i)),
            pl.BlockSpec((D, H1), lambda i: (0, 0)),
        ],
        out_specs=pl.BlockSpec((blk, H1), lambda i: (i, 0)),
        out_shape=jax.ShapeDtypeStruct((rows, H1), jnp.float32),
        compiler_params=pltpu.CompilerParams(
            dimension_semantics=("parallel",),
            vmem_limit_bytes=64 * 1024 * 1024,
        ),
    )(table_t, w)


def _gather_sc(ids, ptab):
    mesh = plsc.VectorSubcoreMesh(core_axis_name="c", subcore_axis_name="s")

    @functools.partial(
        pl.kernel,
        mesh=mesh,
        out_type=jax.ShapeDtypeStruct((B, H1), jnp.float32),
        scratch_types=[
            pltpu.VMEM((_BPW,), jnp.int32),
            pltpu.VMEM((_BPW, H1), jnp.float32),
            pltpu.SemaphoreType.DMA,
        ],
    )
    def gather(idx_hbm, tab_hbm, out_hbm, idx_v, rows_v, sem):
        wid = lax.axis_index("s") * _NC + lax.axis_index("c")
        base = wid * _BPW
        pltpu.sync_copy(idx_hbm.at[pl.ds(base, _BPW)], idx_v)

        def body(g, carry):
            row = g * 16
            vec = idx_v[pl.ds(row, 16)]
            for k in range(16):
                pltpu.async_copy(tab_hbm.at[vec[k]], rows_v.at[row + k], sem)
            return carry

        lax.fori_loop(0, _BPW // 16, body, 0)
        pltpu.make_async_copy(tab_hbm.at[pl.ds(0, _BPW)], rows_v, sem).wait()
        pltpu.sync_copy(rows_v, out_hbm.at[pl.ds(base, _BPW)])

    return gather(ids, ptab)


def _tail_body(u_ref, j_ref, b1_ref, w2_ref, b2_ref,
               w3_ref, b3_ref, w4_ref, b4_ref, o_ref):
    x = jnp.maximum(u_ref[...] + j_ref[...] + b1_ref[...], 0.0)
    x = jnp.maximum(x @ w2_ref[...] + b2_ref[...], 0.0)
    x = jnp.maximum(x @ w3_ref[...] + b3_ref[...], 0.0)
    x = jnp.maximum(x @ w4_ref[...] + b4_ref[...], 0.0)
    o_ref[...] = x.T


def _mlp_tail(gu, gj, b1, W2, b2, W3, b3, W4, b4):
    blk = 16384
    full = lambda s: pl.BlockSpec(s, lambda i: (0, 0))
    return pl.pallas_call(
        _tail_body,
        grid=(B // blk,),
        in_specs=[
            pl.BlockSpec((blk, H1), lambda i: (i, 0)),
            pl.BlockSpec((blk, H1), lambda i: (i, 0)),
            full((1, H1)),
            full((H1, 16)), full((1, 16)),
            full((16, 12)), full((1, 12)),
            full((12, 1)), full((1, 1)),
        ],
        out_specs=pl.BlockSpec((1, blk), lambda i: (0, i)),
        out_shape=jax.ShapeDtypeStruct((1, B), jnp.float32),
        compiler_params=pltpu.CompilerParams(
            dimension_semantics=("arbitrary",),
        ),
    )(gu, gj, b1.reshape(1, H1), W2, b2.reshape(1, 16),
      W3, b3.reshape(1, 12), W4, b4.reshape(1, 1))


def kernel(user_ids, joke_ids, user_table, joke_table,
           W1, b1, W2, b2, W3, b3, W4, b4):
    uids = user_ids.reshape(B).astype(jnp.int32)
    jids = joke_ids.reshape(B).astype(jnp.int32)
    pj = _project(joke_table.T, W1[D:])
    gj = _gather_sc(jids, pj)
    pu = _project(user_table.T, W1[:D])
    gu = _gather_sc(uids, pu)
    out = _mlp_tail(gu, gj, b1, W2, b2, W3, b3, W4, b4)
    return out.reshape(B, 1)


# speedup vs baseline: 1.0390x; 1.0045x over previous
import functools

import jax
import jax.numpy as jnp
from jax import lax
from jax.experimental import pallas as pl
from jax.experimental.pallas import tpu as pltpu
from jax.experimental.pallas import tpu_sc as plsc

B = 16384
D = 64
H1 = 32

_info = plsc.get_sparse_core_info()
_NC, _NS = _info.num_cores, _info.num_subcores
_NW = _NC * _NS
_BPW = B // _NW


def _proj_body(t_ref, w_ref, o_ref):
    pt = lax.dot_general(
        w_ref[...], t_ref[...], (((0,), (0,)), ((), ())),
        preferred_element_type=jnp.float32)
    o_ref[...] = pt.T


def _project(table_t, w):
    rows = table_t.shape[1]
    blk = 38912
    return pl.pallas_call(
        _proj_body,
        grid=(pl.cdiv(rows, blk),),
        in_specs=[
            pl.BlockSpec((D, blk), lambda i: (0, i)),
            pl.BlockSpec((D, H1), lambda i: (0, 0)),
        ],
        out_specs=pl.BlockSpec((blk, H1), lambda i: (i, 0)),
        out_shape=jax.ShapeDtypeStruct((rows, H1), jnp.float32),
        compiler_params=pltpu.CompilerParams(
            dimension_semantics=("arbitrary",),
            vmem_limit_bytes=64 * 1024 * 1024,
        ),
    )(table_t, w)


def _gather_sc(ids, ptab):
    mesh = plsc.VectorSubcoreMesh(core_axis_name="c", subcore_axis_name="s")

    @functools.partial(
        pl.kernel,
        mesh=mesh,
        out_type=jax.ShapeDtypeStruct((B, H1), jnp.float32),
        scratch_types=[
            pltpu.VMEM((_BPW,), jnp.int32),
            pltpu.VMEM((_BPW, H1), jnp.float32),
            pltpu.SemaphoreType.DMA,
        ],
    )
    def gather(idx_hbm, tab_hbm, out_hbm, idx_v, rows_v, sem):
        wid = lax.axis_index("s") * _NC + lax.axis_index("c")
        base = wid * _BPW
        pltpu.sync_copy(idx_hbm.at[pl.ds(base, _BPW)], idx_v)

        def body(g, carry):
            row = g * 16
            vec = idx_v[pl.ds(row, 16)]
            for k in range(16):
                pltpu.async_copy(tab_hbm.at[vec[k]], rows_v.at[row + k], sem)
            return carry

        lax.fori_loop(0, _BPW // 16, body, 0)
        pltpu.make_async_copy(tab_hbm.at[pl.ds(0, _BPW)], rows_v, sem).wait()
        pltpu.sync_copy(rows_v, out_hbm.at[pl.ds(base, _BPW)])

    return gather(ids, ptab)


def _tail_body(u_ref, j_ref, b1_ref, w2_ref, b2_ref,
               w3_ref, b3_ref, w4_ref, b4_ref, o_ref):
    x = jnp.maximum(u_ref[...] + j_ref[...] + b1_ref[...], 0.0)
    x = jnp.maximum(x @ w2_ref[...] + b2_ref[...], 0.0)
    x = jnp.maximum(x @ w3_ref[...] + b3_ref[...], 0.0)
    x = jnp.maximum(x @ w4_ref[...] + b4_ref[...], 0.0)
    o_ref[...] = x.T


def _mlp_tail(gu, gj, b1, W2, b2, W3, b3, W4, b4):
    blk = 8192
    full = lambda s: pl.BlockSpec(s, lambda i: (0, 0))
    return pl.pallas_call(
        _tail_body,
        grid=(B // blk,),
        in_specs=[
            pl.BlockSpec((blk, H1), lambda i: (i, 0)),
            pl.BlockSpec((blk, H1), lambda i: (i, 0)),
            full((1, H1)),
            full((H1, 16)), full((1, 16)),
            full((16, 12)), full((1, 12)),
            full((12, 1)), full((1, 1)),
        ],
        out_specs=pl.BlockSpec((1, blk), lambda i: (0, i)),
        out_shape=jax.ShapeDtypeStruct((1, B), jnp.float32),
        compiler_params=pltpu.CompilerParams(
            dimension_semantics=("arbitrary",),
        ),
    )(gu, gj, b1.reshape(1, H1), W2, b2.reshape(1, 16),
      W3, b3.reshape(1, 12), W4, b4.reshape(1, 1))


def kernel(user_ids, joke_ids, user_table, joke_table,
           W1, b1, W2, b2, W3, b3, W4, b4):
    uids = user_ids.reshape(B).astype(jnp.int32)
    jids = joke_ids.reshape(B).astype(jnp.int32)
    pj = _project(joke_table.T, W1[D:])
    gj = _gather_sc(jids, pj)
    pu = _project(user_table.T, W1[:D])
    gu = _gather_sc(uids, pu)
    out = _mlp_tail(gu, gj, b1, W2, b2, W3, b3, W4, b4)
    return out.reshape(B, 1)
